# Initial kernel scaffold; baseline (speedup 1.0000x reference)
import functools

import jax
import jax.numpy as jnp
from jax import lax
from jax.experimental import pallas as pl
from jax.experimental.pallas import tpu as pltpu
from jax.experimental.pallas import tpu_sc as plsc

_ALPHA = 0.15
_K = 10
_N = 10000
_D = 64
_E = 320000
_B = 16384

_NC = 2
_NS = 16

_EDGES_PER_TEC = _E // _NS
_CHUNK = 80
_NCHUNK = _EDGES_PER_TEC // _CHUNK
_ROWS_PER_TEC = _N // _NS
_RBLK = 125
_NRBLK = _ROWS_PER_TEC // _RBLK

_mesh = plsc.VectorSubcoreMesh(core_axis_name="c", subcore_axis_name="s")


@functools.partial(
    pl.kernel,
    out_type=[jax.ShapeDtypeStruct((_N, 16), jnp.float32)] * 2,
    mesh=_mesh,
    scratch_types=[
        pltpu.VMEM((_CHUNK,), jnp.int32),
        pltpu.VMEM((_CHUNK, 16), jnp.float32),
        pltpu.VMEM((_RBLK, 16), jnp.float32),
        pltpu.VMEM_SHARED((_N, 16), jnp.float32),
    ],
)
def _sc_counts(dst_ui, dst_iu, cnt_i, cnt_u, idx_d, onebuf, zbuf, acc):
    c = lax.axis_index("c")
    s = lax.axis_index("s")

    lane = lax.iota(jnp.float32, 16)
    one_row = jnp.where(lane < 1.0, 1.0, 0.0)

    def fill_one(r, carry):
        onebuf[r, :] = one_row
        return carry

    lax.fori_loop(0, _CHUNK, fill_one, 0)

    def fill_z(r, carry):
        zbuf[r, :] = jnp.zeros((16,), jnp.float32)
        return carry

    lax.fori_loop(0, _RBLK, fill_z, 0)

    row0 = s * _ROWS_PER_TEC
    for j in range(_NRBLK):
        pltpu.sync_copy(zbuf, acc.at[pl.ds(row0 + j * _RBLK, _RBLK)])
    plsc.subcore_barrier()

    def count_dir(dst_e):
        base0 = s * _EDGES_PER_TEC

        def step(i, carry):
            b = base0 + i * _CHUNK
            pltpu.sync_copy(dst_e.at[pl.ds(b, _CHUNK)], idx_d)
            pltpu.sync_copy(onebuf, acc.at[idx_d], add=True)
            return carry

        lax.fori_loop(0, _NCHUNK, step, 0)

    @pl.when(c == 0)
    def _():
        count_dir(dst_ui)

    @pl.when(c == 1)
    def _():
        count_dir(dst_iu)

    plsc.subcore_barrier()

    def flush(j, carry):
        r0 = row0 + j * _RBLK
        pltpu.sync_copy(acc.at[pl.ds(r0, _RBLK)], zbuf)

        @pl.when(c == 0)
        def _():
            pltpu.sync_copy(zbuf, cnt_i.at[pl.ds(r0, _RBLK)])

        @pl.when(c == 1)
        def _():
            pltpu.sync_copy(zbuf, cnt_u.at[pl.ds(r0, _RBLK)])

        return carry

    lax.fori_loop(0, _NRBLK, flush, 0)


@functools.partial(
    pl.kernel,
    out_type=[jax.ShapeDtypeStruct((_N, _D), jnp.float32)] * 2,
    mesh=_mesh,
    scratch_types=[
        pltpu.VMEM((_CHUNK,), jnp.int32),
        pltpu.VMEM((_CHUNK,), jnp.int32),
        pltpu.VMEM((_CHUNK, _D), jnp.float32),
        pltpu.VMEM((_RBLK, _D), jnp.float32),
        pltpu.VMEM((_RBLK, _D), jnp.float32),
        pltpu.VMEM((_RBLK, _D), jnp.float32),
        pltpu.SemaphoreType.DMA,
        pltpu.VMEM_SHARED((_N, _D), jnp.float32),
    ],
)
def _sc_prop(src_ui, dst_ui, src_iu, dst_iu, h_u, h_i, inv_i, inv_u, ah_i,
             ah_u, hi_out, hu_out, idx_s, idx_d, rows, ubuf, ibuf, abuf, sem,
             acc):
    c = lax.axis_index("c")
    s = lax.axis_index("s")
    row0 = s * _ROWS_PER_TEC

    def zrow(r, carry):
        for q in range(_D // 16):
            ubuf[r, pl.ds(16 * q, 16)] = jnp.zeros((16,), jnp.float32)
        return carry

    lax.fori_loop(0, _RBLK, zrow, 0)
    for j in range(_NRBLK):
        pltpu.sync_copy(ubuf, acc.at[pl.ds(row0 + j * _RBLK, _RBLK)])
    plsc.subcore_barrier()

    def direction(src_e, dst_e, h_src, inv, ah, h_out):
        base0 = s * _EDGES_PER_TEC

        def step(i, carry):
            b = base0 + i * _CHUNK
            pltpu.sync_copy(src_e.at[pl.ds(b, _CHUNK)], idx_s)
            pltpu.sync_copy(dst_e.at[pl.ds(b, _CHUNK)], idx_d)
            pltpu.async_copy(h_src.at[idx_s], rows, sem).wait()
            pltpu.sync_copy(rows, acc.at[idx_d], add=True)
            return carry

        lax.fori_loop(0, _NCHUNK, step, 0)
        plsc.subcore_barrier()

        def upd(j, carry):
            r0 = row0 + j * _RBLK
            pltpu.sync_copy(acc.at[pl.ds(r0, _RBLK)], ubuf)
            pltpu.sync_copy(inv.at[pl.ds(r0, _RBLK)], ibuf)
            pltpu.sync_copy(ah.at[pl.ds(r0, _RBLK)], abuf)

            def urow(r, carry2):
                for q in range(_D // 16):
                    sl = pl.ds(16 * q, 16)
                    ubuf[r, sl] = ubuf[r, sl] * ibuf[r, sl] + abuf[r, sl]
                return carry2

            lax.fori_loop(0, _RBLK, urow, 0)
            pltpu.sync_copy(ubuf, h_out.at[pl.ds(r0, _RBLK)])
            return carry

        lax.fori_loop(0, _NRBLK, upd, 0)

    @pl.when(c == 0)
    def _():
        direction(src_ui, dst_ui, h_u, inv_i, ah_i, hi_out)

    @pl.when(c == 1)
    def _():
        direction(src_iu, dst_iu, h_i, inv_u, ah_u, hu_out)


_GB_PER_W = _B // (_NC * _NS)
_GCHUNK = 128
_GN = _GB_PER_W // _GCHUNK


@functools.partial(
    pl.kernel,
    out_type=[jax.ShapeDtypeStruct((_B, _D), jnp.float32)] * 3,
    mesh=_mesh,
    scratch_types=[
        pltpu.VMEM((_GCHUNK,), jnp.int32),
        pltpu.VMEM((_GCHUNK, _D), jnp.float32),
        pltpu.SemaphoreType.DMA,
    ],
)
def _sc_batch_gather(h_u, h_i, user_idx, pos_idx, neg_idx, zu, zp, zn, idx_v,
                     rows, sem):
    c = lax.axis_index("c")
    s = lax.axis_index("s")
    w = s * _NC + c
    base0 = w * _GB_PER_W

    def one(idx_arr, h_arr, out_arr):
        def step(i, carry):
            b = base0 + i * _GCHUNK
            pltpu.sync_copy(idx_arr.at[pl.ds(b, _GCHUNK)], idx_v)
            pltpu.async_copy(h_arr.at[idx_v], rows, sem).wait()
            pltpu.sync_copy(rows, out_arr.at[pl.ds(b, _GCHUNK)])
            return carry

        lax.fori_loop(0, _GN, step, 0)

    one(user_idx, h_u, zu)
    one(pos_idx, h_i, zp)
    one(neg_idx, h_i, zn)


def _tc_ln(x, g, b):
    m = jnp.mean(x, axis=-1, keepdims=True)
    v = jnp.mean((x - m) ** 2, axis=-1, keepdims=True)
    return (x - m) * jax.lax.rsqrt(v + 1e-5) * g + b


def _tc_elu(x):
    return jnp.where(x > 0, x, jnp.exp(jnp.minimum(x, 0.0)) - 1.0)


def _enc_body(x_ref, w1, b1, g1, be1, w2, b2, g2, be2, w3, b3, h0_ref, ah_ref):
    x = x_ref[...]
    h = _tc_elu(_tc_ln(jnp.dot(x, w1[...], preferred_element_type=jnp.float32)
                       + b1[...], g1[...], be1[...]))
    h = _tc_elu(_tc_ln(jnp.dot(h, w2[...], preferred_element_type=jnp.float32)
                       + b2[...], g2[...], be2[...]))
    h0 = jnp.dot(h, w3[...], preferred_element_type=jnp.float32) + b3[...]
    h0_ref[...] = h0
    ah_ref[...] = _ALPHA * h0


def _tc_encode(x, p):
    return pl.pallas_call(
        _enc_body,
        out_shape=[jax.ShapeDtypeStruct((_N, _D), jnp.float32)] * 2,
    )(x, p['W1'], p['b1'].reshape(1, -1), p['g1'].reshape(1, -1),
      p['be1'].reshape(1, -1), p['W2'], p['b2'].reshape(1, -1),
      p['g2'].reshape(1, -1), p['be2'].reshape(1, -1), p['W3'],
      p['b3'].reshape(1, -1))


def _prep_body(cnt_i_ref, cnt_u_ref, inv_i_ref, inv_u_ref):
    ci = cnt_i_ref[:, 0:1]
    cu = cnt_u_ref[:, 0:1]
    inv_i_ref[...] = jnp.broadcast_to(
        (1.0 - _ALPHA) / jnp.maximum(ci, 1.0), (_N, _D))
    inv_u_ref[...] = jnp.broadcast_to(
        (1.0 - _ALPHA) / jnp.maximum(cu, 1.0), (_N, _D))


def _tc_prep(cnt_i, cnt_u):
    return pl.pallas_call(
        _prep_body,
        out_shape=[jax.ShapeDtypeStruct((_N, _D), jnp.float32)] * 2,
    )(cnt_i, cnt_u)


def _pred_body(zu_ref, zp_ref, zn_ref, w1a, w1b, b1, w2, b2, w3, b3, out_ref):
    zu = zu_ref[...]

    def score(z_other):
        h = jnp.dot(zu, w1a[...], preferred_element_type=jnp.float32) \
            + jnp.dot(z_other, w1b[...], preferred_element_type=jnp.float32) \
            + b1[...]
        h = jnp.maximum(h, 0.0)
        h = jnp.maximum(
            jnp.dot(h, w2[...], preferred_element_type=jnp.float32) + b2[...],
            0.0)
        return jnp.dot(h, w3[...], preferred_element_type=jnp.float32) + b3[...]

    d = score(zp_ref[...]) - score(zn_ref[...])
    sp = jnp.maximum(-d, 0.0) + jnp.log(1.0 + jnp.exp(-jnp.abs(d)))
    out_ref[0, 0] = jnp.sum(sp) / _B


def _tc_predict(zu, zp, zn, p):
    out = pl.pallas_call(
        _pred_body,
        out_shape=jax.ShapeDtypeStruct((1, 1), jnp.float32),
    )(zu, zp, zn, p['Wp1'][:_D], p['Wp1'][_D:], p['bp1'].reshape(1, -1),
      p['Wp2'], p['bp2'].reshape(1, -1), p['Wp3'], p['bp3'].reshape(1, -1))
    return out[0, 0]


def kernel(x_user, x_item, params, edge_index_user_item, edge_index_item_user,
           user_idx, pos_idx, neg_idx):
    src_ui = edge_index_user_item[0].astype(jnp.int32)
    dst_ui = edge_index_user_item[1].astype(jnp.int32)
    src_iu = edge_index_item_user[0].astype(jnp.int32)
    dst_iu = edge_index_item_user[1].astype(jnp.int32)

    h0u, ah0u = _tc_encode(x_user, params['user'])
    h0i, ah0i = _tc_encode(x_item, params['item'])

    cnt_i, cnt_u = _sc_counts(dst_ui, dst_iu)
    inv_i, inv_u = _tc_prep(cnt_i, cnt_u)

    hu, hi = h0u, h0i
    for _ in range(_K):
        hi, hu = _sc_prop(src_ui, dst_ui, src_iu, dst_iu, hu, hi, inv_i,
                          inv_u, ah0i, ah0u)

    zu, zp, zn = _sc_batch_gather(hu, hi,
                                  user_idx.astype(jnp.int32),
                                  pos_idx.astype(jnp.int32),
                                  neg_idx.astype(jnp.int32))
    return _tc_predict(zu, zp, zn, params['pred'])


# speedup vs baseline: 4.4075x; 4.4075x over previous
import functools

import jax
import jax.numpy as jnp
from jax import lax
from jax.experimental import pallas as pl
from jax.experimental.pallas import tpu as pltpu
from jax.experimental.pallas import tpu_sc as plsc

_ALPHA = 0.15
_K = 10
_N = 10000
_NP = 10240
_D = 64
_E = 320000
_B = 16384

_NC = 2
_NS = 16

_EDGES_PER_TEC = _E // _NS
_CHUNK = 80
_NCHUNK = _EDGES_PER_TEC // _CHUNK
_ROWS_PER_TEC = _NP // _NS
_RBLK = 128
_NRBLK = _ROWS_PER_TEC // _RBLK

_mesh = plsc.VectorSubcoreMesh(core_axis_name="c", subcore_axis_name="s")


@functools.partial(
    pl.kernel,
    out_type=[jax.ShapeDtypeStruct((_NP, 16), jnp.float32)] * 2,
    mesh=_mesh,
    compiler_params=pltpu.CompilerParams(use_tc_tiling_on_sc=False),
    scratch_types=[
        pltpu.VMEM((_CHUNK,), jnp.int32),
        pltpu.VMEM((_CHUNK, 16), jnp.float32),
        pltpu.VMEM((_RBLK, 16), jnp.float32),
        pltpu.VMEM_SHARED((_NP, 16), jnp.float32),
    ],
)
def _sc_counts(dst_ui, dst_iu, cnt_i, cnt_u, idx_d, onebuf, zbuf, acc):
    c = lax.axis_index("c")
    s = lax.axis_index("s")

    lane = lax.iota(jnp.int32, 16)
    one_row = jnp.where(lane < 1, 1.0, 0.0).astype(jnp.float32)

    def fill_one(r, carry):
        onebuf[r, :] = one_row
        return carry

    lax.fori_loop(0, _CHUNK, fill_one, 0)

    def fill_z(r, carry):
        zbuf[r, :] = jnp.zeros((16,), jnp.float32)
        return carry

    lax.fori_loop(0, _RBLK, fill_z, 0)

    row0 = s * _ROWS_PER_TEC
    for j in range(_NRBLK):
        pltpu.sync_copy(zbuf, acc.at[pl.ds(row0 + j * _RBLK, _RBLK)])
    plsc.subcore_barrier()

    def count_dir(dst_e):
        base0 = s * _EDGES_PER_TEC

        def step(i, carry):
            b = base0 + i * _CHUNK
            pltpu.sync_copy(dst_e.at[pl.ds(b, _CHUNK)], idx_d)
            pltpu.sync_copy(onebuf, acc.at[idx_d], add=True)
            return carry

        lax.fori_loop(0, _NCHUNK, step, 0)

    @pl.when(c == 0)
    def _():
        count_dir(dst_ui)

    @pl.when(c == 1)
    def _():
        count_dir(dst_iu)

    plsc.subcore_barrier()

    def flush(j, carry):
        r0 = row0 + j * _RBLK
        pltpu.sync_copy(acc.at[pl.ds(r0, _RBLK)], zbuf)

        @pl.when(c == 0)
        def _():
            pltpu.sync_copy(zbuf, cnt_i.at[pl.ds(r0, _RBLK)])

        @pl.when(c == 1)
        def _():
            pltpu.sync_copy(zbuf, cnt_u.at[pl.ds(r0, _RBLK)])

        return carry

    lax.fori_loop(0, _NRBLK, flush, 0)


@functools.partial(
    pl.kernel,
    out_type=[jax.ShapeDtypeStruct((_NP, _D), jnp.float32)] * 2,
    mesh=_mesh,
    compiler_params=pltpu.CompilerParams(use_tc_tiling_on_sc=False),
    scratch_types=[
        pltpu.VMEM((_CHUNK,), jnp.int32),
        pltpu.VMEM((_CHUNK,), jnp.int32),
        pltpu.VMEM((_CHUNK, _D), jnp.float32),
        pltpu.VMEM((_RBLK, _D), jnp.float32),
        pltpu.VMEM((_RBLK, _D), jnp.float32),
        pltpu.VMEM((_RBLK, _D), jnp.float32),
        pltpu.SemaphoreType.DMA,
        pltpu.VMEM_SHARED((_NP, _D), jnp.float32),
    ],
)
def _sc_prop(src_ui, dst_ui, src_iu, dst_iu, h_u, h_i, inv_i, inv_u, ah_i,
             ah_u, hi_out, hu_out, idx_s, idx_d, rows, ubuf, ibuf, abuf, sem,
             acc):
    c = lax.axis_index("c")
    s = lax.axis_index("s")
    row0 = s * _ROWS_PER_TEC

    def zrow(r, carry):
        for q in range(_D // 16):
            ubuf[r, pl.ds(16 * q, 16)] = jnp.zeros((16,), jnp.float32)
        return carry

    lax.fori_loop(0, _RBLK, zrow, 0)
    for j in range(_NRBLK):
        pltpu.sync_copy(ubuf, acc.at[pl.ds(row0 + j * _RBLK, _RBLK)])
    plsc.subcore_barrier()

    def direction(src_e, dst_e, h_src, inv, ah, h_out):
        base0 = s * _EDGES_PER_TEC

        def step(i, carry):
            b = base0 + i * _CHUNK
            pltpu.sync_copy(src_e.at[pl.ds(b, _CHUNK)], idx_s)
            pltpu.sync_copy(dst_e.at[pl.ds(b, _CHUNK)], idx_d)
            pltpu.async_copy(h_src.at[idx_s], rows, sem).wait()
            pltpu.sync_copy(rows, acc.at[idx_d], add=True)
            return carry

        lax.fori_loop(0, _NCHUNK, step, 0)
        plsc.subcore_barrier()

        def upd(j, carry):
            r0 = row0 + j * _RBLK
            pltpu.sync_copy(acc.at[pl.ds(r0, _RBLK)], ubuf)
            pltpu.sync_copy(inv.at[pl.ds(r0, _RBLK)], ibuf)
            pltpu.sync_copy(ah.at[pl.ds(r0, _RBLK)], abuf)

            def urow(r, carry2):
                for q in range(_D // 16):
                    sl = pl.ds(16 * q, 16)
                    ubuf[r, sl] = ubuf[r, sl] * ibuf[r, sl] + abuf[r, sl]
                return carry2

            lax.fori_loop(0, _RBLK, urow, 0)
            pltpu.sync_copy(ubuf, h_out.at[pl.ds(r0, _RBLK)])
            return carry

        lax.fori_loop(0, _NRBLK, upd, 0)

    @pl.when(c == 0)
    def _():
        direction(src_ui, dst_ui, h_u, inv_i, ah_i, hi_out)

    @pl.when(c == 1)
    def _():
        direction(src_iu, dst_iu, h_i, inv_u, ah_u, hu_out)


_GB_PER_W = _B // (_NC * _NS)
_GCHUNK = 128
_GN = _GB_PER_W // _GCHUNK


@functools.partial(
    pl.kernel,
    out_type=[jax.ShapeDtypeStruct((_B, _D), jnp.float32)] * 3,
    mesh=_mesh,
    compiler_params=pltpu.CompilerParams(use_tc_tiling_on_sc=False),
    scratch_types=[
        pltpu.VMEM((_GCHUNK,), jnp.int32),
        pltpu.VMEM((_GCHUNK, _D), jnp.float32),
        pltpu.SemaphoreType.DMA,
    ],
)
def _sc_batch_gather(h_u, h_i, user_idx, pos_idx, neg_idx, zu, zp, zn, idx_v,
                     rows, sem):
    c = lax.axis_index("c")
    s = lax.axis_index("s")
    w = s * _NC + c
    base0 = w * _GB_PER_W

    def one(idx_arr, h_arr, out_arr):
        def step(i, carry):
            b = base0 + i * _GCHUNK
            pltpu.sync_copy(idx_arr.at[pl.ds(b, _GCHUNK)], idx_v)
            pltpu.async_copy(h_arr.at[idx_v], rows, sem).wait()
            pltpu.sync_copy(rows, out_arr.at[pl.ds(b, _GCHUNK)])
            return carry

        lax.fori_loop(0, _GN, step, 0)

    one(user_idx, h_u, zu)
    one(pos_idx, h_i, zp)
    one(neg_idx, h_i, zn)


def _tc_ln(x, g, b):
    m = jnp.mean(x, axis=-1, keepdims=True)
    v = jnp.mean((x - m) ** 2, axis=-1, keepdims=True)
    return (x - m) * jax.lax.rsqrt(v + 1e-5) * g + b


def _tc_elu(x):
    return jnp.where(x > 0, x, jnp.exp(jnp.minimum(x, 0.0)) - 1.0)


def _enc_body(x_ref, w1, b1, g1, be1, w2, b2, g2, be2, w3, b3, h0_ref, ah_ref):
    x = x_ref[...]
    h = _tc_elu(_tc_ln(jnp.dot(x, w1[...], preferred_element_type=jnp.float32)
                       + b1[...], g1[...], be1[...]))
    h = _tc_elu(_tc_ln(jnp.dot(h, w2[...], preferred_element_type=jnp.float32)
                       + b2[...], g2[...], be2[...]))
    h0 = jnp.dot(h, w3[...], preferred_element_type=jnp.float32) + b3[...]
    h0_ref[0:_N, :] = h0
    h0_ref[_N:_NP, :] = jnp.zeros((_NP - _N, _D), jnp.float32)
    ah_ref[0:_N, :] = _ALPHA * h0
    ah_ref[_N:_NP, :] = jnp.zeros((_NP - _N, _D), jnp.float32)


def _tc_encode(x, p):
    return pl.pallas_call(
        _enc_body,
        out_shape=[jax.ShapeDtypeStruct((_NP, _D), jnp.float32)] * 2,
    )(x, p['W1'], p['b1'].reshape(1, -1), p['g1'].reshape(1, -1),
      p['be1'].reshape(1, -1), p['W2'], p['b2'].reshape(1, -1),
      p['g2'].reshape(1, -1), p['be2'].reshape(1, -1), p['W3'],
      p['b3'].reshape(1, -1))


def _prep_body(cnt_i_ref, cnt_u_ref, inv_i_ref, inv_u_ref):
    ci = cnt_i_ref[:, 0:1]
    cu = cnt_u_ref[:, 0:1]
    inv_i_ref[...] = jnp.broadcast_to(
        (1.0 - _ALPHA) / jnp.maximum(ci, 1.0), (_NP, _D))
    inv_u_ref[...] = jnp.broadcast_to(
        (1.0 - _ALPHA) / jnp.maximum(cu, 1.0), (_NP, _D))


def _tc_prep(cnt_i, cnt_u):
    return pl.pallas_call(
        _prep_body,
        out_shape=[jax.ShapeDtypeStruct((_NP, _D), jnp.float32)] * 2,
    )(cnt_i, cnt_u)


def _pred_body(zu_ref, zp_ref, zn_ref, w1a, w1b, b1, w2, b2, w3, b3, out_ref):
    zu = zu_ref[...]

    def score(z_other):
        h = jnp.dot(zu, w1a[...], preferred_element_type=jnp.float32) \
            + jnp.dot(z_other, w1b[...], preferred_element_type=jnp.float32) \
            + b1[...]
        h = jnp.maximum(h, 0.0)
        h = jnp.maximum(
            jnp.dot(h, w2[...], preferred_element_type=jnp.float32) + b2[...],
            0.0)
        return jnp.dot(h, w3[...], preferred_element_type=jnp.float32) + b3[...]

    d = score(zp_ref[...]) - score(zn_ref[...])
    sp = jnp.maximum(-d, 0.0) + jnp.log(1.0 + jnp.exp(-jnp.abs(d)))
    out_ref[...] = jnp.reshape(jnp.sum(sp) / _B, (1, 1))


def _tc_predict(zu, zp, zn, p):
    out = pl.pallas_call(
        _pred_body,
        out_shape=jax.ShapeDtypeStruct((1, 1), jnp.float32),
    )(zu, zp, zn, p['Wp1'][:_D], p['Wp1'][_D:], p['bp1'].reshape(1, -1),
      p['Wp2'], p['bp2'].reshape(1, -1), p['Wp3'], p['bp3'].reshape(1, -1))
    return out[0, 0]


def kernel(x_user, x_item, params, edge_index_user_item, edge_index_item_user,
           user_idx, pos_idx, neg_idx):
    src_ui = edge_index_user_item[0].astype(jnp.int32)
    dst_ui = edge_index_user_item[1].astype(jnp.int32)
    src_iu = edge_index_item_user[0].astype(jnp.int32)
    dst_iu = edge_index_item_user[1].astype(jnp.int32)

    h0u, ah0u = _tc_encode(x_user, params['user'])
    h0i, ah0i = _tc_encode(x_item, params['item'])

    cnt_i, cnt_u = _sc_counts(dst_ui, dst_iu)
    inv_i, inv_u = _tc_prep(cnt_i, cnt_u)

    hu, hi = h0u, h0i
    for _ in range(_K):
        hi, hu = _sc_prop(src_ui, dst_ui, src_iu, dst_iu, hu, hi, inv_i,
                          inv_u, ah0i, ah0u)

    zu, zp, zn = _sc_batch_gather(hu, hi,
                                  user_idx.astype(jnp.int32),
                                  pos_idx.astype(jnp.int32),
                                  neg_idx.astype(jnp.int32))
    return _tc_predict(zu, zp, zn, params['pred'])


# speedup vs baseline: 10.6042x; 2.4060x over previous
import functools

import jax
import jax.numpy as jnp
from jax import lax
from jax.experimental import pallas as pl
from jax.experimental.pallas import tpu as pltpu
from jax.experimental.pallas import tpu_sc as plsc

_ALPHA = 0.15
_K = 10
_N = 10000
_NP = 10240
_D = 64
_E = 320000
_B = 16384

_NC = 2
_NS = 16

_EDGES_PER_TEC = _E // _NS
_CHUNK = 80
_NCHUNK = _EDGES_PER_TEC // _CHUNK
_ROWS_PER_TEC = _NP // _NS
_RBLK = 128
_NRBLK = _ROWS_PER_TEC // _RBLK

_mesh = plsc.VectorSubcoreMesh(core_axis_name="c", subcore_axis_name="s")


@functools.partial(
    pl.kernel,
    out_type=[jax.ShapeDtypeStruct((_NP, 16), jnp.float32)] * 2,
    mesh=_mesh,
    compiler_params=pltpu.CompilerParams(use_tc_tiling_on_sc=False),
    scratch_types=[
        pltpu.VMEM((158, 128), jnp.int32),
        pltpu.VMEM((128, 16), jnp.float32),
        pltpu.VMEM((128, 16), jnp.float32),
        pltpu.VMEM_SHARED((_NP, 16), jnp.float32),
    ],
)
def _sc_counts(dst3_ui, dst3_iu, cnt_i, cnt_u, dst2d, onebuf, zbuf, acc):
    c = lax.axis_index("c")
    s = lax.axis_index("s")

    lane = lax.iota(jnp.int32, 16)
    one_row = jnp.where(lane < 1, 1.0, 0.0).astype(jnp.float32)

    def fill_one(r, carry):
        onebuf[r, :] = one_row
        zbuf[r, :] = jnp.zeros((16,), jnp.float32)
        return carry

    lax.fori_loop(0, 128, fill_one, 0)

    row0 = s * _ROWS_PER_TEC
    for j in range(_NRBLK):
        pltpu.sync_copy(zbuf.at[pl.ds(0, 128)],
                        acc.at[pl.ds(row0 + j * _RBLK, _RBLK)])

    def count_dir(dst3):
        pltpu.sync_copy(dst3.at[s], dst2d)
        plsc.subcore_barrier()

        def step(i, carry):
            pltpu.sync_copy(onebuf, acc.at[dst2d.at[i]], add=True)
            return carry

        lax.fori_loop(0, 158, step, 0)

    @pl.when(c == 0)
    def _():
        count_dir(dst3_ui)

    @pl.when(c == 1)
    def _():
        count_dir(dst3_iu)

    plsc.subcore_barrier()

    def flush(j, carry):
        r0 = row0 + j * _RBLK
        pltpu.sync_copy(acc.at[pl.ds(r0, _RBLK)], zbuf)

        @pl.when(c == 0)
        def _():
            pltpu.sync_copy(zbuf, cnt_i.at[pl.ds(r0, _RBLK)])

        @pl.when(c == 1)
        def _():
            pltpu.sync_copy(zbuf, cnt_u.at[pl.ds(r0, _RBLK)])

        return carry

    lax.fori_loop(0, _NRBLK, flush, 0)


_CH = 128
_NCH = 158
_EPAD = _NCH * _CH


@functools.partial(
    pl.kernel,
    out_type=[jax.ShapeDtypeStruct((_NP, _D), jnp.float32)] * 2,
    mesh=_mesh,
    compiler_params=pltpu.CompilerParams(use_tc_tiling_on_sc=False),
    scratch_types=[
        pltpu.VMEM((_NCH, _CH), jnp.int32),
        pltpu.VMEM((_NCH, _CH), jnp.int32),
        pltpu.VMEM((_CH, _D), jnp.float32),
        pltpu.VMEM((_CH, _D), jnp.float32),
        pltpu.VMEM((_RBLK, _D), jnp.float32),
        pltpu.VMEM((_RBLK, _D), jnp.float32),
        pltpu.VMEM((_RBLK, _D), jnp.float32),
        pltpu.SemaphoreType.DMA,
        pltpu.SemaphoreType.DMA,
        pltpu.VMEM_SHARED((_NP, _D), jnp.float32),
    ],
)
def _sc_prop(src3_ui, dst3_ui, src3_iu, dst3_iu, h_u, h_i, inv_i, inv_u, ah_i,
             ah_u, hi_out, hu_out, src2d, dst2d, rows0, rows1, ubuf, ibuf,
             abuf, sem0, sem1, acc):
    c = lax.axis_index("c")
    s = lax.axis_index("s")
    row0 = s * _ROWS_PER_TEC

    def zrow(r, carry):
        for q in range(_D // 16):
            ubuf[r, pl.ds(16 * q, 16)] = jnp.zeros((16,), jnp.float32)
        return carry

    lax.fori_loop(0, _RBLK, zrow, 0)
    for j in range(_NRBLK):
        pltpu.sync_copy(ubuf, acc.at[pl.ds(row0 + j * _RBLK, _RBLK)])

    def direction(src3, dst3, h_src, inv, ah, h_out):
        pltpu.sync_copy(src3.at[s], src2d)
        pltpu.sync_copy(dst3.at[s], dst2d)
        pltpu.async_copy(h_src.at[src2d.at[0]], rows0, sem0)
        pltpu.async_copy(h_src.at[src2d.at[1]], rows1, sem1)
        plsc.subcore_barrier()

        def step(i, carry):
            pltpu.make_async_copy(h_src.at[src2d.at[0]], rows0, sem0).wait()
            pltpu.sync_copy(rows0, acc.at[dst2d.at[2 * i]], add=True)
            pltpu.async_copy(h_src.at[src2d.at[2 * i + 2]], rows0, sem0)
            pltpu.make_async_copy(h_src.at[src2d.at[0]], rows1, sem1).wait()
            pltpu.sync_copy(rows1, acc.at[dst2d.at[2 * i + 1]], add=True)
            pltpu.async_copy(h_src.at[src2d.at[2 * i + 3]], rows1, sem1)
            return carry

        lax.fori_loop(0, _NCH // 2 - 1, step, 0)
        pltpu.make_async_copy(h_src.at[src2d.at[0]], rows0, sem0).wait()
        pltpu.sync_copy(rows0, acc.at[dst2d.at[_NCH - 2]], add=True)
        pltpu.make_async_copy(h_src.at[src2d.at[0]], rows1, sem1).wait()
        pltpu.sync_copy(rows1, acc.at[dst2d.at[_NCH - 1]], add=True)
        plsc.subcore_barrier()

        def upd(j, carry):
            r0 = row0 + j * _RBLK
            pltpu.sync_copy(acc.at[pl.ds(r0, _RBLK)], ubuf)
            pltpu.sync_copy(inv.at[pl.ds(r0, _RBLK)], ibuf)
            pltpu.sync_copy(ah.at[pl.ds(r0, _RBLK)], abuf)

            def urow(r, carry2):
                for q in range(_D // 16):
                    sl = pl.ds(16 * q, 16)
                    ubuf[r, sl] = ubuf[r, sl] * ibuf[r, sl] + abuf[r, sl]
                return carry2

            lax.fori_loop(0, _RBLK, urow, 0)
            pltpu.sync_copy(ubuf, h_out.at[pl.ds(r0, _RBLK)])
            return carry

        lax.fori_loop(0, _NRBLK, upd, 0)

    @pl.when(c == 0)
    def _():
        direction(src3_ui, dst3_ui, h_u, inv_i, ah_i, hi_out)

    @pl.when(c == 1)
    def _():
        direction(src3_iu, dst3_iu, h_i, inv_u, ah_u, hu_out)


_GB_PER_W = _B // (_NC * _NS)
_GCHUNK = 128
_GN = _GB_PER_W // _GCHUNK


@functools.partial(
    pl.kernel,
    out_type=[jax.ShapeDtypeStruct((_B, _D), jnp.float32)] * 3,
    mesh=_mesh,
    compiler_params=pltpu.CompilerParams(use_tc_tiling_on_sc=False),
    scratch_types=[
        pltpu.VMEM((_GCHUNK,), jnp.int32),
        pltpu.VMEM((_GCHUNK, _D), jnp.float32),
        pltpu.SemaphoreType.DMA,
    ],
)
def _sc_batch_gather(h_u, h_i, user_idx, pos_idx, neg_idx, zu, zp, zn, idx_v,
                     rows, sem):
    c = lax.axis_index("c")
    s = lax.axis_index("s")
    w = s * _NC + c
    base0 = w * _GB_PER_W

    def one(idx_arr, h_arr, out_arr):
        def step(i, carry):
            b = base0 + i * _GCHUNK
            pltpu.sync_copy(idx_arr.at[pl.ds(b, _GCHUNK)], idx_v)
            pltpu.async_copy(h_arr.at[idx_v], rows, sem).wait()
            pltpu.sync_copy(rows, out_arr.at[pl.ds(b, _GCHUNK)])
            return carry

        lax.fori_loop(0, _GN, step, 0)

    one(user_idx, h_u, zu)
    one(pos_idx, h_i, zp)
    one(neg_idx, h_i, zn)


def _tc_ln(x, g, b):
    m = jnp.mean(x, axis=-1, keepdims=True)
    v = jnp.mean((x - m) ** 2, axis=-1, keepdims=True)
    return (x - m) * jax.lax.rsqrt(v + 1e-5) * g + b


def _tc_elu(x):
    return jnp.where(x > 0, x, jnp.exp(jnp.minimum(x, 0.0)) - 1.0)


def _enc_body(x_ref, w1, b1, g1, be1, w2, b2, g2, be2, w3, b3, h0_ref, ah_ref):
    x = x_ref[...]
    h = _tc_elu(_tc_ln(jnp.dot(x, w1[...], preferred_element_type=jnp.float32)
                       + b1[...], g1[...], be1[...]))
    h = _tc_elu(_tc_ln(jnp.dot(h, w2[...], preferred_element_type=jnp.float32)
                       + b2[...], g2[...], be2[...]))
    h0 = jnp.dot(h, w3[...], preferred_element_type=jnp.float32) + b3[...]
    h0_ref[0:_N, :] = h0
    h0_ref[_N:_NP, :] = jnp.zeros((_NP - _N, _D), jnp.float32)
    ah_ref[0:_N, :] = _ALPHA * h0
    ah_ref[_N:_NP, :] = jnp.zeros((_NP - _N, _D), jnp.float32)


def _tc_encode(x, p):
    return pl.pallas_call(
        _enc_body,
        out_shape=[jax.ShapeDtypeStruct((_NP, _D), jnp.float32)] * 2,
    )(x, p['W1'], p['b1'].reshape(1, -1), p['g1'].reshape(1, -1),
      p['be1'].reshape(1, -1), p['W2'], p['b2'].reshape(1, -1),
      p['g2'].reshape(1, -1), p['be2'].reshape(1, -1), p['W3'],
      p['b3'].reshape(1, -1))


def _prep_body(cnt_i_ref, cnt_u_ref, inv_i_ref, inv_u_ref):
    ci = cnt_i_ref[:, 0:1]
    cu = cnt_u_ref[:, 0:1]
    inv_i_ref[...] = jnp.broadcast_to(
        (1.0 - _ALPHA) / jnp.maximum(ci, 1.0), (_NP, _D))
    inv_u_ref[...] = jnp.broadcast_to(
        (1.0 - _ALPHA) / jnp.maximum(cu, 1.0), (_NP, _D))


def _tc_prep(cnt_i, cnt_u):
    return pl.pallas_call(
        _prep_body,
        out_shape=[jax.ShapeDtypeStruct((_NP, _D), jnp.float32)] * 2,
    )(cnt_i, cnt_u)


def _pred_body(zu_ref, zp_ref, zn_ref, w1a, w1b, b1, w2, b2, w3, b3, out_ref):
    zu = zu_ref[...]

    def score(z_other):
        h = jnp.dot(zu, w1a[...], preferred_element_type=jnp.float32) \
            + jnp.dot(z_other, w1b[...], preferred_element_type=jnp.float32) \
            + b1[...]
        h = jnp.maximum(h, 0.0)
        h = jnp.maximum(
            jnp.dot(h, w2[...], preferred_element_type=jnp.float32) + b2[...],
            0.0)
        return jnp.dot(h, w3[...], preferred_element_type=jnp.float32) + b3[...]

    d = score(zp_ref[...]) - score(zn_ref[...])
    sp = jnp.maximum(-d, 0.0) + jnp.log(1.0 + jnp.exp(-jnp.abs(d)))
    out_ref[...] = jnp.reshape(jnp.sum(sp) / _B, (1, 1))


def _tc_predict(zu, zp, zn, p):
    out = pl.pallas_call(
        _pred_body,
        out_shape=jax.ShapeDtypeStruct((1, 1), jnp.float32),
    )(zu, zp, zn, p['Wp1'][:_D], p['Wp1'][_D:], p['bp1'].reshape(1, -1),
      p['Wp2'], p['bp2'].reshape(1, -1), p['Wp3'], p['bp3'].reshape(1, -1))
    return out[0, 0]


def kernel(x_user, x_item, params, edge_index_user_item, edge_index_item_user,
           user_idx, pos_idx, neg_idx):
    def slab(a, pad_val):
        a2 = a.astype(jnp.int32).reshape(_NS, _EDGES_PER_TEC)
        pad = jnp.full((_NS, _EPAD - _EDGES_PER_TEC), pad_val, jnp.int32)
        return jnp.concatenate([a2, pad], axis=1).reshape(_NS, _NCH, _CH)

    src3_ui = slab(edge_index_user_item[0], 0)
    dst3_ui = slab(edge_index_user_item[1], _N)
    src3_iu = slab(edge_index_item_user[0], 0)
    dst3_iu = slab(edge_index_item_user[1], _N)

    h0u, ah0u = _tc_encode(x_user, params['user'])
    h0i, ah0i = _tc_encode(x_item, params['item'])

    cnt_i, cnt_u = _sc_counts(dst3_ui, dst3_iu)
    inv_i, inv_u = _tc_prep(cnt_i, cnt_u)

    hu, hi = h0u, h0i
    for _ in range(_K):
        hi, hu = _sc_prop(src3_ui, dst3_ui, src3_iu, dst3_iu, hu, hi, inv_i,
                          inv_u, ah0i, ah0u)

    zu, zp, zn = _sc_batch_gather(hu, hi,
                                  user_idx.astype(jnp.int32),
                                  pos_idx.astype(jnp.int32),
                                  neg_idx.astype(jnp.int32))
    return _tc_predict(zu, zp, zn, params['pred'])
